# baseline (speedup 1.0000x reference)
import jax
import jax.numpy as jnp
from jax import lax
from jax.experimental import pallas as pl
from jax.experimental.pallas import tpu as pltpu
from jax.experimental.pallas import tpu_sc as plsc

N = 10000
E = 320000
D = 128
H = 128
GW = 144
NPOS = 10048

NC, NS, LN = 2, 16, 16
NW = NC * NS
NPH = 2
E2 = E // NPH
SLOTS = 327680
PADP = SLOTS - 2 * E2
PW = SLOTS // NW
RING = 4
CHG = 64
NCHG = PW // CHG
HCH = NCHG // 2
NQ = 2

NPAD = 10240
TPW = NPAD // NS
CHS = 64
NCHS = PW // CHS
QS = NCHS // NQ

BN1 = 2000
BE = 1280
NB_E = E2 // BE
BN5 = 2000

_SC_MESH = plsc.VectorSubcoreMesh(
    core_axis_name="c", subcore_axis_name="s", num_cores=NC, num_subcores=NS)


def _silu(x):
    t = 0.5 * x
    return t + t * jnp.tanh(t)


def _build_table(nf, W1a):
    def body(nf_ref, w_ref, out_ref):
        out_ref[...] = jnp.dot(nf_ref[...], w_ref[...],
                               preferred_element_type=jnp.float32)

    return pl.pallas_call(
        body,
        grid=(N // BN1,),
        in_specs=[pl.BlockSpec((BN1, D), lambda i: (i, 0)),
                  pl.BlockSpec((D, D), lambda i: (0, 0))],
        out_specs=pl.BlockSpec((BN1, D), lambda i: (i, 0)),
        out_shape=jax.ShapeDtypeStruct((N, D), jnp.float32),
    )(nf, W1a)


def _sc_gather(table, posq, idx_src3, idx_oth3):
    def body(table_ref, posq_ref, idxs_ref, idxo_ref, out_ref, rd_ref,
             idx_v, idx_o, r0, r1, r2, r3, pos_v, rb0, rb1,
             s0, s1, s2, s3, t0, t1, t2, t3, u0, u1):
        rows = (r0, r1, r2, r3)
        sems = (s0, s1, s2, s3)
        ssem = (t0, t1, t2, t3)
        rbs = (rb0, rb1)
        rsem = (u0, u1)
        wid = lax.axis_index("c") * NS + lax.axis_index("s")
        pltpu.sync_copy(posq_ref, pos_v)

        lane = lax.iota(jnp.int32, 16)

        def run_half(h):
            base = wid * PW + h * HCH * CHG
            pltpu.sync_copy(idxs_ref.at[wid, pl.ds(h * HCH, HCH)], idx_v)
            pltpu.sync_copy(idxo_ref.at[wid, pl.ds(h * HCH, HCH)], idx_o)

            def out_p(t):
                return out_ref.at[pl.ds(base + t * CHG, CHG)]

            def out_r(t):
                return rd_ref.at[pl.ds(base + t * CHG, CHG)]

            def rdist_chunk(t, p, rb_wait):
                if rb_wait:
                    pltpu.make_async_copy(rbs[p], out_r(t), rsem[p]).wait()
                for l in range(CHG // LN):
                    si = idx_v[t, pl.ds(l * LN, LN)] * 3
                    oi = idx_o[t, pl.ds(l * LN, LN)] * 3
                    zz = jnp.zeros((LN,), jnp.int32)
                    dx = (plsc.load_gather(pos_v, [si])
                          - plsc.load_gather(pos_v, [oi]))
                    dy = (plsc.load_gather(pos_v, [si + 1])
                          - plsc.load_gather(pos_v, [oi + 1]))
                    dz = (plsc.load_gather(pos_v, [si + 2])
                          - plsc.load_gather(pos_v, [oi + 2]))
                    rd = dx * dx + dy * dy + dz * dz
                    plsc.store_scatter(rbs[p], [lane + (l * LN), zz], rd)
                pltpu.async_copy(rbs[p], out_r(t), rsem[p])

            def chunk(t, b, ahead, st_wait, rb_wait):
                rdist_chunk(t, b % 2, rb_wait)
                if ahead:
                    bf = b ^ 2
                    if st_wait:
                        pltpu.make_async_copy(rows[bf], out_p(t),
                                              ssem[bf]).wait()
                    pltpu.async_copy(table_ref.at[idx_v.at[t + 2]],
                                     rows[bf], sems[bf])
                pltpu.make_async_copy(table_ref.at[idx_v.at[t]], rows[b],
                                      sems[b]).wait()
                pltpu.async_copy(rows[b], out_p(t), ssem[b])

            pltpu.async_copy(table_ref.at[idx_v.at[0]], rows[0], sems[0])
            pltpu.async_copy(table_ref.at[idx_v.at[1]], rows[1], sems[1])
            chunk(0, 0, True, False, False)
            chunk(1, 1, True, False, False)
            chunk(2, 2, True, True, True)
            chunk(3, 3, True, True, True)

            def grp(g, c2):
                t = g * RING
                for b in range(RING):
                    chunk(t + b, b, True, True, True)
                return c2

            lax.fori_loop(1, HCH // RING - 1, grp, 0)
            tl = HCH - RING
            chunk(tl, 0, True, True, True)
            chunk(tl + 1, 1, True, True, True)
            chunk(tl + 2, 2, False, False, True)
            chunk(tl + 3, 3, False, False, True)

            for b in range(RING):
                pltpu.make_async_copy(rows[b], out_p(tl + b), ssem[b]).wait()
            for p in range(2):
                pltpu.make_async_copy(rbs[p], out_r(tl + 2 + p),
                                      rsem[p]).wait()

        run_half(0)
        run_half(1)

    f = pl.kernel(
        body,
        out_type=(jax.ShapeDtypeStruct((SLOTS, D), jnp.float32),
                  jax.ShapeDtypeStruct((SLOTS, LN), jnp.float32)),
        mesh=_SC_MESH,
        compiler_params=pltpu.CompilerParams(needs_layout_passes=False),
        scratch_types=[
            pltpu.VMEM((HCH, CHG), jnp.int32),
            pltpu.VMEM((HCH, CHG), jnp.int32),
            pltpu.VMEM((CHG, D), jnp.float32),
            pltpu.VMEM((CHG, D), jnp.float32),
            pltpu.VMEM((CHG, D), jnp.float32),
            pltpu.VMEM((CHG, D), jnp.float32),
            pltpu.VMEM((NPOS * 3,), jnp.float32),
            pltpu.VMEM((CHG, LN), jnp.float32),
            pltpu.VMEM((CHG, LN), jnp.float32),
            pltpu.SemaphoreType.DMA,
            pltpu.SemaphoreType.DMA,
            pltpu.SemaphoreType.DMA,
            pltpu.SemaphoreType.DMA,
            pltpu.SemaphoreType.DMA,
            pltpu.SemaphoreType.DMA,
            pltpu.SemaphoreType.DMA,
            pltpu.SemaphoreType.DMA,
            pltpu.SemaphoreType.DMA,
            pltpu.SemaphoreType.DMA,
        ],
    )
    return f(table, posq, idx_src3, idx_oth3)


def _edge_mlp(gathered, rdq, ef, W1b_pad, b1r, w1c, W2, b2r, pofs):
    def body(g_ref, r_ref, ef_ref, w1b_ref, b1_ref, w1c_ref, w2_ref, b2_ref,
             out_ref):
        gs = g_ref[...]
        rd = r_ref[...][:, 0:1]
        c = lax.dot_general(ef_ref[...], w1b_ref[...],
                            (((0,), (0,)), ((), ())),
                            preferred_element_type=jnp.float32) + b1_ref[...]
        h1 = _silu(gs + c + rd * w1c_ref[...])
        out_ref[...] = _silu(
            jnp.dot(h1.astype(jnp.bfloat16), w2_ref[...],
                    preferred_element_type=jnp.float32)
            + b2_ref[...])

    return pl.pallas_call(
        body,
        grid=(NB_E, 2),
        in_specs=[
            pl.BlockSpec((BE, D), lambda i, d: (d * NB_E + i, 0)),
            pl.BlockSpec((BE, LN), lambda i, d: (d * NB_E + i, 0)),
            pl.BlockSpec((16, BE), lambda i, d: (0, pofs + i)),
            pl.BlockSpec((16, H), lambda i, d: (0, 0)),
            pl.BlockSpec((1, H), lambda i, d: (0, 0)),
            pl.BlockSpec((1, H), lambda i, d: (0, 0)),
            pl.BlockSpec((H, H), lambda i, d: (0, 0)),
            pl.BlockSpec((1, H), lambda i, d: (0, 0)),
        ],
        out_specs=pl.BlockSpec((BE, H), lambda i, d: (d * NB_E + i, 0)),
        out_shape=jax.ShapeDtypeStruct((SLOTS, H), jnp.float32),
    )(gathered, rdq, ef, W1b_pad, b1r, w1c, W2, b2r)


def _sc_scatter(msgs, idx3, zrow):
    def body(msgs_ref, idx_ref, zrow_ref, out_ref,
             idx_v, m0, m1, m2, m3, acc, s0, s1, s2, s3):
        bufs = (m0, m1, m2, m3)
        sems = (s0, s1, s2, s3)
        cid = lax.axis_index("c")
        sid = lax.axis_index("s")
        wid = cid * NS + sid

        pltpu.async_copy(zrow_ref, m0, s0)
        pltpu.make_async_copy(zrow_ref, m0, s0).wait()
        for k in range(TPW // CHS):
            pltpu.sync_copy(m0, acc.at[pl.ds(sid * TPW + k * CHS, CHS)])
        plsc.subcore_barrier()

        base = wid * PW

        def quarter(q, carry):
            pltpu.sync_copy(idx_ref.at[wid, pl.ds(q * QS, QS)], idx_v)
            qbase = base + q * QS * CHS
            for b in range(RING):
                pltpu.async_copy(
                    msgs_ref.at[pl.ds(qbase + b * CHS, CHS)], bufs[b],
                    sems[b])

            def chunk(jq, b, fire):
                pltpu.make_async_copy(
                    msgs_ref.at[pl.ds(qbase + jq * CHS, CHS)], bufs[b],
                    sems[b]).wait()
                pltpu.sync_copy(bufs[b], acc.at[idx_v.at[jq]], add=True)
                if fire:
                    pltpu.async_copy(
                        msgs_ref.at[pl.ds(qbase + (jq + RING) * CHS, CHS)],
                        bufs[b], sems[b])

            def grp(g, c2):
                for b in range(RING):
                    chunk(g * RING + b, b, True)
                return c2

            lax.fori_loop(0, QS // RING - 1, grp, 0)
            for b in range(RING):
                chunk((QS // RING - 1) * RING + b, b, False)
            return carry

        lax.fori_loop(0, NQ, quarter, 0)
        plsc.subcore_barrier()

        pltpu.sync_copy(acc.at[pl.ds(sid * TPW, TPW)],
                        out_ref.at[pl.ds(cid * NPAD + sid * TPW, TPW)])

    f = pl.kernel(
        body,
        out_type=jax.ShapeDtypeStruct((NC * NPAD, H), jnp.float32),
        mesh=_SC_MESH,
        compiler_params=pltpu.CompilerParams(needs_layout_passes=False),
        scratch_types=[
            pltpu.VMEM((QS, CHS), jnp.int32),
            pltpu.VMEM((CHS, H), jnp.float32),
            pltpu.VMEM((CHS, H), jnp.float32),
            pltpu.VMEM((CHS, H), jnp.float32),
            pltpu.VMEM((CHS, H), jnp.float32),
            pltpu.VMEM_SHARED((NPAD, H), jnp.float32),
            pltpu.SemaphoreType.DMA,
            pltpu.SemaphoreType.DMA,
            pltpu.SemaphoreType.DMA,
            pltpu.SemaphoreType.DMA,
        ],
    )
    return f(msgs, idx3, zrow)


def _node_update(nf, parts, U1a, U1b, ub1r, U2, ub2r, gammar, betar):
    def body(nf_ref, a0_ref, a1_ref, a2_ref, a3_ref, u1a_ref, u1b_ref,
             ub1_ref, u2_ref, ub2_ref, g_ref, be_ref, out_ref):
        x0 = nf_ref[...]
        agg = ((a0_ref[...] + a1_ref[...]) + (a2_ref[...] + a3_ref[...]))
        h = _silu(jnp.dot(x0, u1a_ref[...], preferred_element_type=jnp.float32)
                  + jnp.dot(agg, u1b_ref[...],
                            preferred_element_type=jnp.float32)
                  + ub1_ref[...])
        x = x0 + jnp.dot(h, u2_ref[...], preferred_element_type=jnp.float32) \
            + ub2_ref[...]
        mu = jnp.mean(x, axis=1, keepdims=True)
        xc = x - mu
        var = jnp.mean(xc * xc, axis=1, keepdims=True)
        out_ref[...] = xc * lax.rsqrt(var + 1e-5) * g_ref[...] + be_ref[...]

    a_spec = pl.BlockSpec((BN5, H), lambda i: (i, 0))
    return pl.pallas_call(
        body,
        grid=(N // BN5,),
        in_specs=[
            pl.BlockSpec((BN5, D), lambda i: (i, 0)),
            a_spec, a_spec, a_spec, a_spec,
            pl.BlockSpec((D, H), lambda i: (0, 0)),
            pl.BlockSpec((H, H), lambda i: (0, 0)),
            pl.BlockSpec((1, H), lambda i: (0, 0)),
            pl.BlockSpec((H, D), lambda i: (0, 0)),
            pl.BlockSpec((1, D), lambda i: (0, 0)),
            pl.BlockSpec((1, D), lambda i: (0, 0)),
            pl.BlockSpec((1, D), lambda i: (0, 0)),
        ],
        out_specs=pl.BlockSpec((BN5, D), lambda i: (i, 0)),
        out_shape=jax.ShapeDtypeStruct((N, D), jnp.float32),
    )(nf, *parts, U1a, U1b, ub1r, U2, ub2r, gammar, betar)


def kernel(node_features, edge_features, edge_index, positions,
           W1, b1, W2, b2, U1, ub1, U2, ub2, ln_gamma, ln_beta):
    row = edge_index[0]
    col = edge_index[1]
    spread = (jnp.arange(PADP, dtype=jnp.int32) * 509) % N
    dump = jnp.full((PADP,), N, jnp.int32)

    W1a = W1[:D]
    W1b_pad = jnp.concatenate(
        [jnp.zeros((3, H), jnp.float32), W1[D:D + 13]], axis=0)
    w1c = W1[D + 13].reshape(1, H)
    zrow = jnp.zeros((CHS, H), jnp.float32)
    posq = jnp.zeros((NPOS, 3), jnp.float32).at[:N].set(
        positions).reshape(NPOS * 3)
    eft = edge_features.T

    table = _build_table(node_features, W1a)

    parts = []
    for p in range(NPH):
        lo, hi = p * E2, (p + 1) * E2
        rp, cp = row[lo:hi], col[lo:hi]
        idx_src = jnp.concatenate([cp, rp, spread]).reshape(NW, NCHG, CHG)
        idx_oth = jnp.concatenate([rp, cp, spread]).reshape(NW, NCHG, CHG)
        idx_sct = jnp.concatenate([rp, cp, dump]).reshape(NW, NCHS, CHS)
        g_p, rd_p = _sc_gather(table, posq, idx_src, idx_oth)
        m_p = _edge_mlp(g_p, rd_p, eft, W1b_pad, b1.reshape(1, H),
                        w1c, W2.astype(jnp.bfloat16), b2.reshape(1, H),
                        p * NB_E)
        pp = _sc_scatter(m_p, idx_sct, zrow)
        parts.append(pp[:N])
        parts.append(pp[NPAD:NPAD + N])

    out = _node_update(node_features, parts, U1[:D], U1[D:],
                       ub1.reshape(1, H), U2, ub2.reshape(1, D),
                       ln_gamma.reshape(1, D), ln_beta.reshape(1, D))
    return out


# speedup vs baseline: 1.1448x; 1.1448x over previous
import jax
import jax.numpy as jnp
from jax import lax
from jax.experimental import pallas as pl
from jax.experimental.pallas import tpu as pltpu
from jax.experimental.pallas import tpu_sc as plsc

N = 10000
E = 320000
D = 128
H = 128
GW = 144
NPOS = 10048

NC, NS, LN = 2, 16, 16
NW = NC * NS
NPH = 2
E2 = E // NPH
SLOTS = 327680
PADP = SLOTS - 2 * E2
PW = SLOTS // NW
RING = 4
CHG = 64
NCHG = PW // CHG
HCH = NCHG // 2
NQ = 2

NPAD = 10240
TPW = NPAD // NS
CHS = 64
NCHS = PW // CHS
QS = NCHS // NQ

BN1 = 2000
BE = 3200
NB_E = E2 // BE
BN5 = 2000

_SC_MESH = plsc.VectorSubcoreMesh(
    core_axis_name="c", subcore_axis_name="s", num_cores=NC, num_subcores=NS)


def _silu(x):
    t = 0.5 * x
    return t + t * jnp.tanh(t)


def _build_table(nf, W1a):
    def body(nf_ref, w_ref, out_ref):
        out_ref[...] = jnp.dot(nf_ref[...], w_ref[...],
                               preferred_element_type=jnp.float32)

    return pl.pallas_call(
        body,
        grid=(N // BN1,),
        in_specs=[pl.BlockSpec((BN1, D), lambda i: (i, 0)),
                  pl.BlockSpec((D, D), lambda i: (0, 0))],
        out_specs=pl.BlockSpec((BN1, D), lambda i: (i, 0)),
        out_shape=jax.ShapeDtypeStruct((N, D), jnp.float32),
    )(nf, W1a)


def _sc_gather(table, posq, idx_src3, idx_oth3):
    def body(table_ref, posq_ref, idxs_ref, idxo_ref, out_ref, rd_ref,
             idx_v, idx_o, r0, r1, r2, r3, pos_v, rb0, rb1,
             s0, s1, s2, s3, t0, t1, t2, t3, u0, u1):
        rows = (r0, r1, r2, r3)
        sems = (s0, s1, s2, s3)
        ssem = (t0, t1, t2, t3)
        rbs = (rb0, rb1)
        rsem = (u0, u1)
        wid = lax.axis_index("c") * NS + lax.axis_index("s")
        pltpu.sync_copy(posq_ref, pos_v)

        lane = lax.iota(jnp.int32, 16)

        def run_half(h):
            base = wid * PW + h * HCH * CHG
            pltpu.sync_copy(idxs_ref.at[wid, pl.ds(h * HCH, HCH)], idx_v)
            pltpu.sync_copy(idxo_ref.at[wid, pl.ds(h * HCH, HCH)], idx_o)

            def out_p(t):
                return out_ref.at[pl.ds(base + t * CHG, CHG)]

            def out_r(t):
                return rd_ref.at[pl.ds(base + t * CHG, CHG)]

            def rdist_chunk(t, p, rb_wait):
                if rb_wait:
                    pltpu.make_async_copy(rbs[p], out_r(t), rsem[p]).wait()
                for l in range(CHG // LN):
                    si = idx_v[t, pl.ds(l * LN, LN)] * 3
                    oi = idx_o[t, pl.ds(l * LN, LN)] * 3
                    zz = jnp.zeros((LN,), jnp.int32)
                    dx = (plsc.load_gather(pos_v, [si])
                          - plsc.load_gather(pos_v, [oi]))
                    dy = (plsc.load_gather(pos_v, [si + 1])
                          - plsc.load_gather(pos_v, [oi + 1]))
                    dz = (plsc.load_gather(pos_v, [si + 2])
                          - plsc.load_gather(pos_v, [oi + 2]))
                    rd = dx * dx + dy * dy + dz * dz
                    plsc.store_scatter(rbs[p], [lane + (l * LN), zz], rd)
                pltpu.async_copy(rbs[p], out_r(t), rsem[p])

            def chunk(t, b, ahead, st_wait, rb_wait):
                rdist_chunk(t, b % 2, rb_wait)
                if ahead:
                    bf = b ^ 2
                    if st_wait:
                        pltpu.make_async_copy(rows[bf], out_p(t),
                                              ssem[bf]).wait()
                    pltpu.async_copy(table_ref.at[idx_v.at[t + 2]],
                                     rows[bf], sems[bf])
                pltpu.make_async_copy(table_ref.at[idx_v.at[t]], rows[b],
                                      sems[b]).wait()
                pltpu.async_copy(rows[b], out_p(t), ssem[b])

            pltpu.async_copy(table_ref.at[idx_v.at[0]], rows[0], sems[0])
            pltpu.async_copy(table_ref.at[idx_v.at[1]], rows[1], sems[1])
            chunk(0, 0, True, False, False)
            chunk(1, 1, True, False, False)
            chunk(2, 2, True, True, True)
            chunk(3, 3, True, True, True)

            def grp(g, c2):
                t = g * RING
                for b in range(RING):
                    chunk(t + b, b, True, True, True)
                return c2

            lax.fori_loop(1, HCH // RING - 1, grp, 0)
            tl = HCH - RING
            chunk(tl, 0, True, True, True)
            chunk(tl + 1, 1, True, True, True)
            chunk(tl + 2, 2, False, False, True)
            chunk(tl + 3, 3, False, False, True)

            for b in range(RING):
                pltpu.make_async_copy(rows[b], out_p(tl + b), ssem[b]).wait()
            for p in range(2):
                pltpu.make_async_copy(rbs[p], out_r(tl + 2 + p),
                                      rsem[p]).wait()

        run_half(0)
        run_half(1)

    f = pl.kernel(
        body,
        out_type=(jax.ShapeDtypeStruct((SLOTS, D), jnp.float32),
                  jax.ShapeDtypeStruct((SLOTS, LN), jnp.float32)),
        mesh=_SC_MESH,
        compiler_params=pltpu.CompilerParams(needs_layout_passes=False),
        scratch_types=[
            pltpu.VMEM((HCH, CHG), jnp.int32),
            pltpu.VMEM((HCH, CHG), jnp.int32),
            pltpu.VMEM((CHG, D), jnp.float32),
            pltpu.VMEM((CHG, D), jnp.float32),
            pltpu.VMEM((CHG, D), jnp.float32),
            pltpu.VMEM((CHG, D), jnp.float32),
            pltpu.VMEM((NPOS * 3,), jnp.float32),
            pltpu.VMEM((CHG, LN), jnp.float32),
            pltpu.VMEM((CHG, LN), jnp.float32),
            pltpu.SemaphoreType.DMA,
            pltpu.SemaphoreType.DMA,
            pltpu.SemaphoreType.DMA,
            pltpu.SemaphoreType.DMA,
            pltpu.SemaphoreType.DMA,
            pltpu.SemaphoreType.DMA,
            pltpu.SemaphoreType.DMA,
            pltpu.SemaphoreType.DMA,
            pltpu.SemaphoreType.DMA,
            pltpu.SemaphoreType.DMA,
        ],
    )
    return f(table, posq, idx_src3, idx_oth3)


def _edge_mlp(gathered, rdq, ef, W1b_pad, b1r, w1c, W2, b2r, pofs):
    def body(g_ref, r_ref, ef_ref, w1b_ref, b1_ref, w1c_ref, w2_ref, b2_ref,
             out_ref):
        gs = g_ref[...]
        rd = r_ref[...][:, 0:1]
        c = lax.dot_general(ef_ref[...], w1b_ref[...],
                            (((0,), (0,)), ((), ())),
                            preferred_element_type=jnp.float32) + b1_ref[...]
        h1 = _silu(gs + c + rd * w1c_ref[...])
        out_ref[...] = _silu(
            jnp.dot(h1, w2_ref[...], preferred_element_type=jnp.float32)
            + b2_ref[...])

    return pl.pallas_call(
        body,
        grid=(NB_E, 2),
        in_specs=[
            pl.BlockSpec((BE, D), lambda i, d: (d * NB_E + i, 0)),
            pl.BlockSpec((BE, LN), lambda i, d: (d * NB_E + i, 0)),
            pl.BlockSpec((16, BE), lambda i, d: (0, pofs + i)),
            pl.BlockSpec((16, H), lambda i, d: (0, 0)),
            pl.BlockSpec((1, H), lambda i, d: (0, 0)),
            pl.BlockSpec((1, H), lambda i, d: (0, 0)),
            pl.BlockSpec((H, H), lambda i, d: (0, 0)),
            pl.BlockSpec((1, H), lambda i, d: (0, 0)),
        ],
        out_specs=pl.BlockSpec((BE, H), lambda i, d: (d * NB_E + i, 0)),
        out_shape=jax.ShapeDtypeStruct((SLOTS, H), jnp.float32),
    )(gathered, rdq, ef, W1b_pad, b1r, w1c, W2, b2r)


def _sc_scatter(msgs, idx3, zrow):
    def body(msgs_ref, idx_ref, zrow_ref, out_ref,
             idx_v, m0, m1, m2, m3, acc, s0, s1, s2, s3):
        bufs = (m0, m1, m2, m3)
        sems = (s0, s1, s2, s3)
        cid = lax.axis_index("c")
        sid = lax.axis_index("s")
        wid = cid * NS + sid

        pltpu.async_copy(zrow_ref, m0, s0)
        pltpu.make_async_copy(zrow_ref, m0, s0).wait()
        for k in range(TPW // CHS):
            pltpu.sync_copy(m0, acc.at[pl.ds(sid * TPW + k * CHS, CHS)])
        plsc.subcore_barrier()

        base = wid * PW

        def quarter(q, carry):
            pltpu.sync_copy(idx_ref.at[wid, pl.ds(q * QS, QS)], idx_v)
            qbase = base + q * QS * CHS
            for b in range(RING):
                pltpu.async_copy(
                    msgs_ref.at[pl.ds(qbase + b * CHS, CHS)], bufs[b],
                    sems[b])

            def chunk(jq, b, fire):
                pltpu.make_async_copy(
                    msgs_ref.at[pl.ds(qbase + jq * CHS, CHS)], bufs[b],
                    sems[b]).wait()
                pltpu.sync_copy(bufs[b], acc.at[idx_v.at[jq]], add=True)
                if fire:
                    pltpu.async_copy(
                        msgs_ref.at[pl.ds(qbase + (jq + RING) * CHS, CHS)],
                        bufs[b], sems[b])

            def grp(g, c2):
                for b in range(RING):
                    chunk(g * RING + b, b, True)
                return c2

            lax.fori_loop(0, QS // RING - 1, grp, 0)
            for b in range(RING):
                chunk((QS // RING - 1) * RING + b, b, False)
            return carry

        lax.fori_loop(0, NQ, quarter, 0)
        plsc.subcore_barrier()

        pltpu.sync_copy(acc.at[pl.ds(sid * TPW, TPW)],
                        out_ref.at[pl.ds(cid * NPAD + sid * TPW, TPW)])

    f = pl.kernel(
        body,
        out_type=jax.ShapeDtypeStruct((NC * NPAD, H), jnp.float32),
        mesh=_SC_MESH,
        compiler_params=pltpu.CompilerParams(needs_layout_passes=False),
        scratch_types=[
            pltpu.VMEM((QS, CHS), jnp.int32),
            pltpu.VMEM((CHS, H), jnp.float32),
            pltpu.VMEM((CHS, H), jnp.float32),
            pltpu.VMEM((CHS, H), jnp.float32),
            pltpu.VMEM((CHS, H), jnp.float32),
            pltpu.VMEM_SHARED((NPAD, H), jnp.float32),
            pltpu.SemaphoreType.DMA,
            pltpu.SemaphoreType.DMA,
            pltpu.SemaphoreType.DMA,
            pltpu.SemaphoreType.DMA,
        ],
    )
    return f(msgs, idx3, zrow)


def _node_update(nf, parts, U1a, U1b, ub1r, U2, ub2r, gammar, betar):
    def body(nf_ref, a0_ref, a1_ref, a2_ref, a3_ref, u1a_ref, u1b_ref,
             ub1_ref, u2_ref, ub2_ref, g_ref, be_ref, out_ref):
        x0 = nf_ref[...]
        agg = ((a0_ref[...] + a1_ref[...]) + (a2_ref[...] + a3_ref[...]))
        h = _silu(jnp.dot(x0, u1a_ref[...], preferred_element_type=jnp.float32)
                  + jnp.dot(agg, u1b_ref[...],
                            preferred_element_type=jnp.float32)
                  + ub1_ref[...])
        x = x0 + jnp.dot(h, u2_ref[...], preferred_element_type=jnp.float32) \
            + ub2_ref[...]
        mu = jnp.mean(x, axis=1, keepdims=True)
        xc = x - mu
        var = jnp.mean(xc * xc, axis=1, keepdims=True)
        out_ref[...] = xc * lax.rsqrt(var + 1e-5) * g_ref[...] + be_ref[...]

    a_spec = pl.BlockSpec((BN5, H), lambda i: (i, 0))
    return pl.pallas_call(
        body,
        grid=(N // BN5,),
        in_specs=[
            pl.BlockSpec((BN5, D), lambda i: (i, 0)),
            a_spec, a_spec, a_spec, a_spec,
            pl.BlockSpec((D, H), lambda i: (0, 0)),
            pl.BlockSpec((H, H), lambda i: (0, 0)),
            pl.BlockSpec((1, H), lambda i: (0, 0)),
            pl.BlockSpec((H, D), lambda i: (0, 0)),
            pl.BlockSpec((1, D), lambda i: (0, 0)),
            pl.BlockSpec((1, D), lambda i: (0, 0)),
            pl.BlockSpec((1, D), lambda i: (0, 0)),
        ],
        out_specs=pl.BlockSpec((BN5, D), lambda i: (i, 0)),
        out_shape=jax.ShapeDtypeStruct((N, D), jnp.float32),
    )(nf, *parts, U1a, U1b, ub1r, U2, ub2r, gammar, betar)


def kernel(node_features, edge_features, edge_index, positions,
           W1, b1, W2, b2, U1, ub1, U2, ub2, ln_gamma, ln_beta):
    row = edge_index[0]
    col = edge_index[1]
    spread = (jnp.arange(PADP, dtype=jnp.int32) * 509) % N
    dump = jnp.full((PADP,), N, jnp.int32)

    W1a = W1[:D]
    W1b_pad = jnp.concatenate(
        [jnp.zeros((3, H), jnp.float32), W1[D:D + 13]], axis=0)
    w1c = W1[D + 13].reshape(1, H)
    zrow = jnp.zeros((CHS, H), jnp.float32)
    posq = jnp.zeros((NPOS, 3), jnp.float32).at[:N].set(
        positions).reshape(NPOS * 3)
    eft = edge_features.T

    table = _build_table(node_features, W1a)

    parts = []
    for p in range(NPH):
        lo, hi = p * E2, (p + 1) * E2
        rp, cp = row[lo:hi], col[lo:hi]
        idx_src = jnp.concatenate([cp, rp, spread]).reshape(NW, NCHG, CHG)
        idx_oth = jnp.concatenate([rp, cp, spread]).reshape(NW, NCHG, CHG)
        idx_sct = jnp.concatenate([rp, cp, dump]).reshape(NW, NCHS, CHS)
        g_p, rd_p = _sc_gather(table, posq, idx_src, idx_oth)
        m_p = _edge_mlp(g_p, rd_p, eft, W1b_pad, b1.reshape(1, H),
                        w1c, W2, b2.reshape(1, H), p * NB_E)
        pp = _sc_scatter(m_p, idx_sct, zrow)
        parts.append(pp[:N])
        parts.append(pp[NPAD:NPAD + N])

    out = _node_update(node_features, parts, U1[:D], U1[D:],
                       ub1.reshape(1, H), U2, ub2.reshape(1, D),
                       ln_gamma.reshape(1, D), ln_beta.reshape(1, D))
    return out


# speedup vs baseline: 1.1714x; 1.0232x over previous
import jax
import jax.numpy as jnp
from jax import lax
from jax.experimental import pallas as pl
from jax.experimental.pallas import tpu as pltpu
from jax.experimental.pallas import tpu_sc as plsc

N = 10000
E = 320000
D = 128
H = 128
GW = 144
NPOS = 10048

NC, NS, LN = 2, 16, 16
NW = NC * NS
NPH = 2
E2 = E // NPH
SLOTS = 327680
PADP = SLOTS - 2 * E2
PW = SLOTS // NW
RING = 4
CHG = 64
NCHG = PW // CHG
HCH = NCHG // 2
NQ = 2

NPAD = 10240
TPW = NPAD // NS
CHS = 64
NCHS = PW // CHS
QS = NCHS // NQ

BN1 = 2000
BE = 6400
NB_E = E2 // BE
BN5 = 2000

_SC_MESH = plsc.VectorSubcoreMesh(
    core_axis_name="c", subcore_axis_name="s", num_cores=NC, num_subcores=NS)


def _silu(x):
    t = 0.5 * x
    return t + t * jnp.tanh(t)


def _build_table(nf, W1a):
    def body(nf_ref, w_ref, out_ref):
        out_ref[...] = jnp.dot(nf_ref[...], w_ref[...],
                               preferred_element_type=jnp.float32)

    return pl.pallas_call(
        body,
        grid=(N // BN1,),
        in_specs=[pl.BlockSpec((BN1, D), lambda i: (i, 0)),
                  pl.BlockSpec((D, D), lambda i: (0, 0))],
        out_specs=pl.BlockSpec((BN1, D), lambda i: (i, 0)),
        out_shape=jax.ShapeDtypeStruct((N, D), jnp.float32),
    )(nf, W1a)


def _sc_gather(table, posq, idx_src3, idx_oth3):
    def body(table_ref, posq_ref, idxs_ref, idxo_ref, out_ref, rd_ref,
             idx_v, idx_o, r0, r1, r2, r3, pos_v, rb0, rb1,
             s0, s1, s2, s3, t0, t1, t2, t3, u0, u1):
        rows = (r0, r1, r2, r3)
        sems = (s0, s1, s2, s3)
        ssem = (t0, t1, t2, t3)
        rbs = (rb0, rb1)
        rsem = (u0, u1)
        wid = lax.axis_index("c") * NS + lax.axis_index("s")
        pltpu.sync_copy(posq_ref, pos_v)

        lane = lax.iota(jnp.int32, 16)

        def run_half(h):
            base = wid * PW + h * HCH * CHG
            pltpu.sync_copy(idxs_ref.at[wid, pl.ds(h * HCH, HCH)], idx_v)
            pltpu.sync_copy(idxo_ref.at[wid, pl.ds(h * HCH, HCH)], idx_o)

            def out_p(t):
                return out_ref.at[pl.ds(base + t * CHG, CHG)]

            def out_r(t):
                return rd_ref.at[pl.ds(base + t * CHG, CHG)]

            def rdist_chunk(t, p, rb_wait):
                if rb_wait:
                    pltpu.make_async_copy(rbs[p], out_r(t), rsem[p]).wait()
                for l in range(CHG // LN):
                    si = idx_v[t, pl.ds(l * LN, LN)] * 3
                    oi = idx_o[t, pl.ds(l * LN, LN)] * 3
                    zz = jnp.zeros((LN,), jnp.int32)
                    dx = (plsc.load_gather(pos_v, [si])
                          - plsc.load_gather(pos_v, [oi]))
                    dy = (plsc.load_gather(pos_v, [si + 1])
                          - plsc.load_gather(pos_v, [oi + 1]))
                    dz = (plsc.load_gather(pos_v, [si + 2])
                          - plsc.load_gather(pos_v, [oi + 2]))
                    rd = dx * dx + dy * dy + dz * dz
                    plsc.store_scatter(rbs[p], [lane + (l * LN), zz], rd)
                pltpu.async_copy(rbs[p], out_r(t), rsem[p])

            def chunk(t, b, ahead, st_wait, rb_wait):
                rdist_chunk(t, b % 2, rb_wait)
                if ahead:
                    bf = b ^ 2
                    if st_wait:
                        pltpu.make_async_copy(rows[bf], out_p(t),
                                              ssem[bf]).wait()
                    pltpu.async_copy(table_ref.at[idx_v.at[t + 2]],
                                     rows[bf], sems[bf])
                pltpu.make_async_copy(table_ref.at[idx_v.at[t]], rows[b],
                                      sems[b]).wait()
                pltpu.async_copy(rows[b], out_p(t), ssem[b])

            pltpu.async_copy(table_ref.at[idx_v.at[0]], rows[0], sems[0])
            pltpu.async_copy(table_ref.at[idx_v.at[1]], rows[1], sems[1])
            chunk(0, 0, True, False, False)
            chunk(1, 1, True, False, False)
            chunk(2, 2, True, True, True)
            chunk(3, 3, True, True, True)

            def grp(g, c2):
                t = g * RING
                for b in range(RING):
                    chunk(t + b, b, True, True, True)
                return c2

            lax.fori_loop(1, HCH // RING - 1, grp, 0)
            tl = HCH - RING
            chunk(tl, 0, True, True, True)
            chunk(tl + 1, 1, True, True, True)
            chunk(tl + 2, 2, False, False, True)
            chunk(tl + 3, 3, False, False, True)

            for b in range(RING):
                pltpu.make_async_copy(rows[b], out_p(tl + b), ssem[b]).wait()
            for p in range(2):
                pltpu.make_async_copy(rbs[p], out_r(tl + 2 + p),
                                      rsem[p]).wait()

        run_half(0)
        run_half(1)

    f = pl.kernel(
        body,
        out_type=(jax.ShapeDtypeStruct((SLOTS, D), jnp.float32),
                  jax.ShapeDtypeStruct((SLOTS, LN), jnp.float32)),
        mesh=_SC_MESH,
        compiler_params=pltpu.CompilerParams(needs_layout_passes=False),
        scratch_types=[
            pltpu.VMEM((HCH, CHG), jnp.int32),
            pltpu.VMEM((HCH, CHG), jnp.int32),
            pltpu.VMEM((CHG, D), jnp.float32),
            pltpu.VMEM((CHG, D), jnp.float32),
            pltpu.VMEM((CHG, D), jnp.float32),
            pltpu.VMEM((CHG, D), jnp.float32),
            pltpu.VMEM((NPOS * 3,), jnp.float32),
            pltpu.VMEM((CHG, LN), jnp.float32),
            pltpu.VMEM((CHG, LN), jnp.float32),
            pltpu.SemaphoreType.DMA,
            pltpu.SemaphoreType.DMA,
            pltpu.SemaphoreType.DMA,
            pltpu.SemaphoreType.DMA,
            pltpu.SemaphoreType.DMA,
            pltpu.SemaphoreType.DMA,
            pltpu.SemaphoreType.DMA,
            pltpu.SemaphoreType.DMA,
            pltpu.SemaphoreType.DMA,
            pltpu.SemaphoreType.DMA,
        ],
    )
    return f(table, posq, idx_src3, idx_oth3)


def _edge_mlp(gathered, rdq, ef, W1b_pad, b1r, w1c, W2, b2r, pofs):
    def body(g_ref, r_ref, ef_ref, w1b_ref, b1_ref, w1c_ref, w2_ref, b2_ref,
             out_ref):
        gs = g_ref[...]
        rd = r_ref[...][:, 0:1]
        c = lax.dot_general(ef_ref[...], w1b_ref[...],
                            (((0,), (0,)), ((), ())),
                            preferred_element_type=jnp.float32) + b1_ref[...]
        h1 = _silu(gs + c + rd * w1c_ref[...])
        out_ref[...] = _silu(
            jnp.dot(h1, w2_ref[...], preferred_element_type=jnp.float32)
            + b2_ref[...])

    return pl.pallas_call(
        body,
        grid=(NB_E, 2),
        in_specs=[
            pl.BlockSpec((BE, D), lambda i, d: (d * NB_E + i, 0)),
            pl.BlockSpec((BE, LN), lambda i, d: (d * NB_E + i, 0)),
            pl.BlockSpec((16, BE), lambda i, d: (0, pofs + i)),
            pl.BlockSpec((16, H), lambda i, d: (0, 0)),
            pl.BlockSpec((1, H), lambda i, d: (0, 0)),
            pl.BlockSpec((1, H), lambda i, d: (0, 0)),
            pl.BlockSpec((H, H), lambda i, d: (0, 0)),
            pl.BlockSpec((1, H), lambda i, d: (0, 0)),
        ],
        out_specs=pl.BlockSpec((BE, H), lambda i, d: (d * NB_E + i, 0)),
        out_shape=jax.ShapeDtypeStruct((SLOTS, H), jnp.float32),
    )(gathered, rdq, ef, W1b_pad, b1r, w1c, W2, b2r)


def _sc_scatter(msgs, idx3, zrow):
    def body(msgs_ref, idx_ref, zrow_ref, out_ref,
             idx_v, m0, m1, m2, m3, acc, s0, s1, s2, s3):
        bufs = (m0, m1, m2, m3)
        sems = (s0, s1, s2, s3)
        cid = lax.axis_index("c")
        sid = lax.axis_index("s")
        wid = cid * NS + sid

        pltpu.async_copy(zrow_ref, m0, s0)
        pltpu.make_async_copy(zrow_ref, m0, s0).wait()
        for k in range(TPW // CHS):
            pltpu.sync_copy(m0, acc.at[pl.ds(sid * TPW + k * CHS, CHS)])
        plsc.subcore_barrier()

        base = wid * PW

        def quarter(q, carry):
            pltpu.sync_copy(idx_ref.at[wid, pl.ds(q * QS, QS)], idx_v)
            qbase = base + q * QS * CHS
            for b in range(RING):
                pltpu.async_copy(
                    msgs_ref.at[pl.ds(qbase + b * CHS, CHS)], bufs[b],
                    sems[b])

            def chunk(jq, b, fire):
                pltpu.make_async_copy(
                    msgs_ref.at[pl.ds(qbase + jq * CHS, CHS)], bufs[b],
                    sems[b]).wait()
                pltpu.sync_copy(bufs[b], acc.at[idx_v.at[jq]], add=True)
                if fire:
                    pltpu.async_copy(
                        msgs_ref.at[pl.ds(qbase + (jq + RING) * CHS, CHS)],
                        bufs[b], sems[b])

            def grp(g, c2):
                for b in range(RING):
                    chunk(g * RING + b, b, True)
                return c2

            lax.fori_loop(0, QS // RING - 1, grp, 0)
            for b in range(RING):
                chunk((QS // RING - 1) * RING + b, b, False)
            return carry

        lax.fori_loop(0, NQ, quarter, 0)
        plsc.subcore_barrier()

        pltpu.sync_copy(acc.at[pl.ds(sid * TPW, TPW)],
                        out_ref.at[pl.ds(cid * NPAD + sid * TPW, TPW)])

    f = pl.kernel(
        body,
        out_type=jax.ShapeDtypeStruct((NC * NPAD, H), jnp.float32),
        mesh=_SC_MESH,
        compiler_params=pltpu.CompilerParams(needs_layout_passes=False),
        scratch_types=[
            pltpu.VMEM((QS, CHS), jnp.int32),
            pltpu.VMEM((CHS, H), jnp.float32),
            pltpu.VMEM((CHS, H), jnp.float32),
            pltpu.VMEM((CHS, H), jnp.float32),
            pltpu.VMEM((CHS, H), jnp.float32),
            pltpu.VMEM_SHARED((NPAD, H), jnp.float32),
            pltpu.SemaphoreType.DMA,
            pltpu.SemaphoreType.DMA,
            pltpu.SemaphoreType.DMA,
            pltpu.SemaphoreType.DMA,
        ],
    )
    return f(msgs, idx3, zrow)


def _node_update(nf, parts, U1a, U1b, ub1r, U2, ub2r, gammar, betar):
    def body(nf_ref, a0_ref, a1_ref, a2_ref, a3_ref, u1a_ref, u1b_ref,
             ub1_ref, u2_ref, ub2_ref, g_ref, be_ref, out_ref):
        x0 = nf_ref[...]
        agg = ((a0_ref[...] + a1_ref[...]) + (a2_ref[...] + a3_ref[...]))
        h = _silu(jnp.dot(x0, u1a_ref[...], preferred_element_type=jnp.float32)
                  + jnp.dot(agg, u1b_ref[...],
                            preferred_element_type=jnp.float32)
                  + ub1_ref[...])
        x = x0 + jnp.dot(h, u2_ref[...], preferred_element_type=jnp.float32) \
            + ub2_ref[...]
        mu = jnp.mean(x, axis=1, keepdims=True)
        xc = x - mu
        var = jnp.mean(xc * xc, axis=1, keepdims=True)
        out_ref[...] = xc * lax.rsqrt(var + 1e-5) * g_ref[...] + be_ref[...]

    a_spec = pl.BlockSpec((BN5, H), lambda i: (i, 0))
    return pl.pallas_call(
        body,
        grid=(N // BN5,),
        in_specs=[
            pl.BlockSpec((BN5, D), lambda i: (i, 0)),
            a_spec, a_spec, a_spec, a_spec,
            pl.BlockSpec((D, H), lambda i: (0, 0)),
            pl.BlockSpec((H, H), lambda i: (0, 0)),
            pl.BlockSpec((1, H), lambda i: (0, 0)),
            pl.BlockSpec((H, D), lambda i: (0, 0)),
            pl.BlockSpec((1, D), lambda i: (0, 0)),
            pl.BlockSpec((1, D), lambda i: (0, 0)),
            pl.BlockSpec((1, D), lambda i: (0, 0)),
        ],
        out_specs=pl.BlockSpec((BN5, D), lambda i: (i, 0)),
        out_shape=jax.ShapeDtypeStruct((N, D), jnp.float32),
    )(nf, *parts, U1a, U1b, ub1r, U2, ub2r, gammar, betar)


def kernel(node_features, edge_features, edge_index, positions,
           W1, b1, W2, b2, U1, ub1, U2, ub2, ln_gamma, ln_beta):
    row = edge_index[0]
    col = edge_index[1]
    spread = (jnp.arange(PADP, dtype=jnp.int32) * 509) % N
    dump = jnp.full((PADP,), N, jnp.int32)

    W1a = W1[:D]
    W1b_pad = jnp.concatenate(
        [jnp.zeros((3, H), jnp.float32), W1[D:D + 13]], axis=0)
    w1c = W1[D + 13].reshape(1, H)
    zrow = jnp.zeros((CHS, H), jnp.float32)
    posq = jnp.zeros((NPOS, 3), jnp.float32).at[:N].set(
        positions).reshape(NPOS * 3)
    eft = edge_features.T

    table = _build_table(node_features, W1a)

    parts = []
    for p in range(NPH):
        lo, hi = p * E2, (p + 1) * E2
        rp, cp = row[lo:hi], col[lo:hi]
        idx_src = jnp.concatenate([cp, rp, spread]).reshape(NW, NCHG, CHG)
        idx_oth = jnp.concatenate([rp, cp, spread]).reshape(NW, NCHG, CHG)
        idx_sct = jnp.concatenate([rp, cp, dump]).reshape(NW, NCHS, CHS)
        g_p, rd_p = _sc_gather(table, posq, idx_src, idx_oth)
        m_p = _edge_mlp(g_p, rd_p, eft, W1b_pad, b1.reshape(1, H),
                        w1c, W2, b2.reshape(1, H), p * NB_E)
        pp = _sc_scatter(m_p, idx_sct, zrow)
        parts.append(pp[:N])
        parts.append(pp[NPAD:NPAD + N])

    out = _node_update(node_features, parts, U1[:D], U1[D:],
                       ub1.reshape(1, H), U2, ub2.reshape(1, D),
                       ln_gamma.reshape(1, D), ln_beta.reshape(1, D))
    return out
